# baseline (speedup 1.0000x reference)
import functools

import jax
import jax.numpy as jnp
from jax.experimental import pallas as pl
from jax.experimental.pallas import tpu as pltpu

_TMP = 0.1
_FMP = 0.1


def _mask_body(tm_ref, fm_ref, x_ref, o_ref):
    x = x_ref[0, 0]
    tm = tm_ref[0]
    fm = fm_ref[0]
    o_ref[0, 0] = x * tm * fm


def kernel(spec):
    B, C, Fd, T = spec.shape
    tlen = int(T * _TMP)
    flen = int(Fd * _FMP)
    t0 = jax.random.randint(
        jax.random.fold_in(jax.random.key(1), 0), (B,), 0, max(1, T - tlen + 1)
    )
    f0 = jax.random.randint(
        jax.random.fold_in(jax.random.key(1), 1), (B,), 0, max(1, Fd - flen + 1)
    )
    tidx = jnp.arange(T)[None, :]
    tm = jnp.where((tidx >= t0[:, None]) & (tidx < (t0 + tlen)[:, None]), 0.0, 1.0)
    fidx = jnp.arange(Fd)[None, :]
    fm = jnp.where((fidx >= f0[:, None]) & (fidx < (f0 + flen)[:, None]), 0.0, 1.0)
    tm = tm.astype(spec.dtype).reshape(B, 1, T)
    fm = fm.astype(spec.dtype).reshape(B, Fd, 1)

    return pl.pallas_call(
        _mask_body,
        grid=(B,),
        in_specs=[
            pl.BlockSpec((1, 1, T), lambda b: (b, 0, 0)),
            pl.BlockSpec((1, Fd, 1), lambda b: (b, 0, 0)),
            pl.BlockSpec((1, C, Fd, T), lambda b: (b, 0, 0, 0)),
        ],
        out_specs=pl.BlockSpec((1, C, Fd, T), lambda b: (b, 0, 0, 0)),
        out_shape=jax.ShapeDtypeStruct(spec.shape, spec.dtype),
    )(tm, fm, spec)


# speedup vs baseline: 1.0216x; 1.0216x over previous
import functools

import jax
import jax.numpy as jnp
from jax.experimental import pallas as pl
from jax.experimental.pallas import tpu as pltpu

_TMP = 0.1
_FMP = 0.1


def _mask_body(tm_ref, fm_ref, x_ref, o_ref):
    o_ref[...] = x_ref[...] * tm_ref[...] * fm_ref[...]


def kernel(spec):
    B, C, Fd, T = spec.shape
    tlen = int(T * _TMP)
    flen = int(Fd * _FMP)
    t0 = jax.random.randint(
        jax.random.fold_in(jax.random.key(1), 0), (B,), 0, max(1, T - tlen + 1)
    )
    f0 = jax.random.randint(
        jax.random.fold_in(jax.random.key(1), 1), (B,), 0, max(1, Fd - flen + 1)
    )
    tidx = jnp.arange(T)[None, :]
    tm = jnp.where((tidx >= t0[:, None]) & (tidx < (t0 + tlen)[:, None]), 0.0, 1.0)
    fidx = jnp.arange(Fd)[None, :]
    fm = jnp.where((fidx >= f0[:, None]) & (fidx < (f0 + flen)[:, None]), 0.0, 1.0)
    tm = tm.astype(spec.dtype).reshape(B, 1, 1, T)
    fm = fm.astype(spec.dtype).reshape(B, 1, Fd, 1)

    BB = 4
    return pl.pallas_call(
        _mask_body,
        grid=(B // BB,),
        in_specs=[
            pl.BlockSpec((BB, 1, 1, T), lambda b: (b, 0, 0, 0)),
            pl.BlockSpec((BB, 1, Fd, 1), lambda b: (b, 0, 0, 0)),
            pl.BlockSpec((BB, C, Fd, T), lambda b: (b, 0, 0, 0)),
        ],
        out_specs=pl.BlockSpec((BB, C, Fd, T), lambda b: (b, 0, 0, 0)),
        out_shape=jax.ShapeDtypeStruct(spec.shape, spec.dtype),
    )(tm, fm, spec)
